# baseline (speedup 1.0000x reference)
import functools

import jax
import jax.numpy as jnp
from jax import lax
from jax.experimental import pallas as pl
from jax.experimental.pallas import tpu as pltpu
from jax.experimental.pallas import tpu_sc as plsc

_B, _F, _V, _E, _NUM = 4096, 26, 100000, 64, 13
_D = _F * _E + _E
_NC, _NS = 2, 16
_NW = _NC * _NS
_RPW = _B * _F // _NW
_J = _RPW // 128
_JP = 32
_HALF = _F * _V // 2
_GB = 10000


def _repack_body(a_ref, b_ref, out_ref):
    out_ref[:, :_E] = a_ref[0]
    out_ref[:, _E:] = b_ref[0]


def _tc_repack(tables):
    return pl.pallas_call(
        _repack_body,
        grid=(_HALF // _GB,),
        in_specs=[
            pl.BlockSpec((1, _GB, _E), lambda i: (i // 10, i % 10, 0)),
            pl.BlockSpec((1, _GB, _E), lambda i: (13 + i // 10, i % 10, 0)),
        ],
        out_specs=pl.BlockSpec((_GB, 2 * _E), lambda i: (i, 0)),
        out_shape=jax.ShapeDtypeStruct((_HALF, 2 * _E), jnp.float32),
    )(tables, tables)


def _make_sc_gather():
    mesh = plsc.VectorSubcoreMesh(
        core_axis_name="c", subcore_axis_name="s",
        num_cores=_NC, num_subcores=_NS)

    @functools.partial(
        pl.kernel,
        out_type=jax.ShapeDtypeStruct((_NW, _J, 128, 128), jnp.float32),
        mesh=mesh,
        scratch_types=[
            pltpu.VMEM((_JP, 128), jnp.int32),
            pltpu.VMEM((128, 128), jnp.float32),
            pltpu.SemaphoreType.DMA,
        ],
    )
    def sc_gather(idx_hbm, tab_hbm, out_hbm, idx_v, rows_v, sem):
        wid = lax.axis_index("s") * _NC + lax.axis_index("c")
        pltpu.sync_copy(idx_hbm.at[wid], idx_v)

        def body(j, carry):
            pltpu.async_copy(tab_hbm.at[idx_v.at[j]], rows_v, sem).wait()
            pltpu.sync_copy(rows_v, out_hbm.at[wid, j])
            return carry

        lax.fori_loop(0, _J, body, 0, unroll=False)

    return sc_gather


_SC_GATHER_CACHE = []


def _sc_gather_fn():
    if not _SC_GATHER_CACHE:
        _SC_GATHER_CACHE.append(_make_sc_gather())
    return _SC_GATHER_CACHE[0]


def _tc_dense_body(g_ref, xn_ref, wn_ref, bn_ref, wc_ref, wnt_ref,
                   bf_ref, out_ref):
    num_emb = lax.dot_general(
        xn_ref[...], wn_ref[...], (((1,), (1,)), ((), ())),
        preferred_element_type=jnp.float32) + bn_ref[...]
    o = bf_ref[...] + lax.dot_general(
        num_emb, wnt_ref[...], (((1,), (0,)), ((), ())),
        preferred_element_type=jnp.float32)
    g = g_ref[...]
    for f in range(_F):
        emb = g[:, f, :_E] if f < 13 else g[:, f, _E:]
        o = o + lax.dot_general(
            emb, wc_ref[f], (((1,), (0,)), ((), ())),
            preferred_element_type=jnp.float32)
    out_ref[...] = o


def _tc_dense(g, X_num, W_num, b_num2, Wc3, WnT, b_final2):
    BB = 512
    return pl.pallas_call(
        _tc_dense_body,
        grid=(_B // BB,),
        in_specs=[
            pl.BlockSpec((BB, _F, 128), lambda i: (i, 0, 0)),
            pl.BlockSpec((BB, _NUM), lambda i: (i, 0)),
            pl.BlockSpec((_E, _NUM), lambda i: (0, 0)),
            pl.BlockSpec((1, _E), lambda i: (0, 0)),
            pl.BlockSpec((_F, _E, _E), lambda i: (0, 0, 0)),
            pl.BlockSpec((_E, _E), lambda i: (0, 0)),
            pl.BlockSpec((1, _E), lambda i: (0, 0)),
        ],
        out_specs=pl.BlockSpec((BB, _E), lambda i: (i, 0)),
        out_shape=jax.ShapeDtypeStruct((_B, _E), jnp.float32),
    )(g, X_num, W_num, b_num2, Wc3, WnT, b_final2)


def kernel(X_cat, X_num, tables, W_num, b_num, W_final, b_final):
    xc = X_cat.astype(jnp.int32)
    fcol = jnp.arange(_F, dtype=jnp.int32) % 13
    pair_idx = (xc + (fcol * _V)[None, :]).reshape(_NW, _J, 128)
    pair_idx = jnp.pad(pair_idx, ((0, 0), (0, _JP - _J), (0, 0)))

    tab2 = _tc_repack(tables)
    gathered = _sc_gather_fn()(pair_idx, tab2)
    g = gathered.reshape(_B, _F, 128)

    Wc3 = jnp.transpose(W_final[:, : _F * _E].reshape(_E, _F, _E), (1, 2, 0))
    WnT = W_final[:, _F * _E:].T
    return _tc_dense(g, X_num, W_num, b_num.reshape(1, _E),
                     Wc3, WnT, b_final.reshape(1, _E))


# speedup vs baseline: 1.0014x; 1.0014x over previous
import functools

import jax
import jax.numpy as jnp
from jax import lax
from jax.experimental import pallas as pl
from jax.experimental.pallas import tpu as pltpu
from jax.experimental.pallas import tpu_sc as plsc

_B, _F, _V, _E, _NUM = 4096, 26, 100000, 64, 13
_D = _F * _E + _E
_NC, _NS = 2, 16
_NW = _NC * _NS
_RPW = _B * _F // _NW
_J = _RPW // 128
_JP = 32
_HALF = _F * _V // 2
_GB = 20000


def _repack_body(a_ref, out_ref):
    out_ref[:, :_E] = a_ref[0]
    out_ref[:, _E:] = a_ref[1]


def _tc_repack(tables):
    nv = _V // _GB
    return pl.pallas_call(
        _repack_body,
        grid=(_HALF // _GB,),
        in_specs=[
            pl.BlockSpec((2, _GB, _E), lambda i: (i // nv, i % nv, 0)),
        ],
        out_specs=pl.BlockSpec((_GB, 2 * _E), lambda i: (i, 0)),
        out_shape=jax.ShapeDtypeStruct((_HALF, 2 * _E), jnp.float32),
    )(tables)


def _make_sc_gather():
    mesh = plsc.VectorSubcoreMesh(
        core_axis_name="c", subcore_axis_name="s",
        num_cores=_NC, num_subcores=_NS)

    @functools.partial(
        pl.kernel,
        out_type=jax.ShapeDtypeStruct((_NW, _J, 128, 128), jnp.float32),
        mesh=mesh,
        scratch_types=[
            pltpu.VMEM((_JP, 128), jnp.int32),
            pltpu.VMEM((128, 128), jnp.float32),
            pltpu.SemaphoreType.DMA,
        ],
    )
    def sc_gather(idx_hbm, tab_hbm, out_hbm, idx_v, rows_v, sem):
        wid = lax.axis_index("s") * _NC + lax.axis_index("c")
        pltpu.sync_copy(idx_hbm.at[wid], idx_v)

        def body(j, carry):
            pltpu.async_copy(tab_hbm.at[idx_v.at[j]], rows_v, sem).wait()
            pltpu.sync_copy(rows_v, out_hbm.at[wid, j])
            return carry

        lax.fori_loop(0, _J, body, 0, unroll=False)

    return sc_gather


_SC_GATHER_CACHE = []


def _sc_gather_fn():
    if not _SC_GATHER_CACHE:
        _SC_GATHER_CACHE.append(_make_sc_gather())
    return _SC_GATHER_CACHE[0]


def _tc_dense_body(g_ref, xn_ref, wn_ref, bn_ref, wc_ref, wnt_ref,
                   bf_ref, out_ref):
    num_emb = lax.dot_general(
        xn_ref[...], wn_ref[...], (((1,), (1,)), ((), ())),
        preferred_element_type=jnp.float32) + bn_ref[...]
    o = bf_ref[...] + lax.dot_general(
        num_emb, wnt_ref[...], (((1,), (0,)), ((), ())),
        preferred_element_type=jnp.float32)
    g = g_ref[...]
    for f in range(_F):
        emb = g[:, f, :_E] if f % 2 == 0 else g[:, f, _E:]
        o = o + lax.dot_general(
            emb, wc_ref[f], (((1,), (0,)), ((), ())),
            preferred_element_type=jnp.float32)
    out_ref[...] = o


def _tc_dense(g, X_num, W_num, b_num2, Wc3, WnT, b_final2):
    BB = 512
    return pl.pallas_call(
        _tc_dense_body,
        grid=(_B // BB,),
        in_specs=[
            pl.BlockSpec((BB, _F, 128), lambda i: (i, 0, 0)),
            pl.BlockSpec((BB, _NUM), lambda i: (i, 0)),
            pl.BlockSpec((_E, _NUM), lambda i: (0, 0)),
            pl.BlockSpec((1, _E), lambda i: (0, 0)),
            pl.BlockSpec((_F, _E, _E), lambda i: (0, 0, 0)),
            pl.BlockSpec((_E, _E), lambda i: (0, 0)),
            pl.BlockSpec((1, _E), lambda i: (0, 0)),
        ],
        out_specs=pl.BlockSpec((BB, _E), lambda i: (i, 0)),
        out_shape=jax.ShapeDtypeStruct((_B, _E), jnp.float32),
    )(g, X_num, W_num, b_num2, Wc3, WnT, b_final2)


def kernel(X_cat, X_num, tables, W_num, b_num, W_final, b_final):
    xc = X_cat.astype(jnp.int32)
    fcol = jnp.arange(_F, dtype=jnp.int32) // 2
    pair_idx = (xc + (fcol * _V)[None, :]).reshape(_NW, _J, 128)
    pair_idx = jnp.pad(pair_idx, ((0, 0), (0, _JP - _J), (0, 0)))

    tab2 = _tc_repack(tables)
    gathered = _sc_gather_fn()(pair_idx, tab2)
    g = gathered.reshape(_B, _F, 128)

    Wc3 = jnp.transpose(W_final[:, : _F * _E].reshape(_E, _F, _E), (1, 2, 0))
    WnT = W_final[:, _F * _E:].T
    return _tc_dense(g, X_num, W_num, b_num.reshape(1, _E),
                     Wc3, WnT, b_final.reshape(1, _E))


# speedup vs baseline: 1.9291x; 1.9265x over previous
import functools

import jax
import jax.numpy as jnp
from jax import lax
from jax.experimental import pallas as pl
from jax.experimental.pallas import tpu as pltpu
from jax.experimental.pallas import tpu_sc as plsc

_B, _F, _V, _E, _NUM = 4096, 26, 100000, 64, 13
_D = _F * _E + _E
_NC, _NS = 2, 16
_NW = _NC * _NS
_RPW = _B * _F // _NW
_J = _RPW // 128
_JP = 32
_GV = 4096
_NV = -(-_V // _GV)
_VP = _NV * _GV


def _transform_body(t_ref, wc_ref, out_ref):
    a = lax.dot_general(
        t_ref[0], wc_ref[0, 0], (((0,), (0,)), ((), ())),
        preferred_element_type=jnp.float32)
    b = lax.dot_general(
        t_ref[1], wc_ref[0, 1], (((0,), (0,)), ((), ())),
        preferred_element_type=jnp.float32)
    out_ref[:, :_E] = a
    out_ref[:, _E:] = b


def _tc_transform(tabT, Wc4):
    return pl.pallas_call(
        _transform_body,
        grid=(13, _NV),
        in_specs=[
            pl.BlockSpec((2, _E, _GV), lambda k, j: (k, 0, j)),
            pl.BlockSpec((1, 2, _E, _E), lambda k, j: (k, 0, 0, 0)),
        ],
        out_specs=pl.BlockSpec((_GV, 2 * _E), lambda k, j: (k * _NV + j, 0)),
        out_shape=jax.ShapeDtypeStruct((13 * _VP, 2 * _E), jnp.float32),
    )(tabT, Wc4)


def _make_sc_gather():
    mesh = plsc.VectorSubcoreMesh(
        core_axis_name="c", subcore_axis_name="s",
        num_cores=_NC, num_subcores=_NS)

    @functools.partial(
        pl.kernel,
        out_type=jax.ShapeDtypeStruct((_NW, _J, 128, 128), jnp.float32),
        mesh=mesh,
        scratch_types=[
            pltpu.VMEM((_JP, 128), jnp.int32),
            pltpu.VMEM((128, 128), jnp.float32),
            pltpu.SemaphoreType.DMA,
        ],
    )
    def sc_gather(idx_hbm, tab_hbm, out_hbm, idx_v, rows_v, sem):
        wid = lax.axis_index("s") * _NC + lax.axis_index("c")
        pltpu.sync_copy(idx_hbm.at[wid], idx_v)

        def body(j, carry):
            pltpu.async_copy(tab_hbm.at[idx_v.at[j]], rows_v, sem).wait()
            pltpu.sync_copy(rows_v, out_hbm.at[wid, j])
            return carry

        lax.fori_loop(0, _J, body, 0, unroll=False)

    return sc_gather


_SC_GATHER_CACHE = []


def _sc_gather_fn():
    if not _SC_GATHER_CACHE:
        _SC_GATHER_CACHE.append(_make_sc_gather())
    return _SC_GATHER_CACHE[0]


def _tc_combine_body(g_ref, xn_ref, wn_ref, bn_ref, wnt_ref, bf_ref, out_ref):
    num_emb = lax.dot_general(
        xn_ref[...], wn_ref[...], (((1,), (1,)), ((), ())),
        preferred_element_type=jnp.float32) + bn_ref[...]
    o = bf_ref[...] + lax.dot_general(
        num_emb, wnt_ref[...], (((1,), (0,)), ((), ())),
        preferred_element_type=jnp.float32)
    g = g_ref[...]
    for f in range(_F):
        o = o + (g[:, f, :_E] if f % 2 == 0 else g[:, f, _E:])
    out_ref[...] = o


def _tc_combine(g, X_num, W_num, b_num2, WnT, b_final2):
    BB = 512
    return pl.pallas_call(
        _tc_combine_body,
        grid=(_B // BB,),
        in_specs=[
            pl.BlockSpec((BB, _F, 128), lambda i: (i, 0, 0)),
            pl.BlockSpec((BB, _NUM), lambda i: (i, 0)),
            pl.BlockSpec((_E, _NUM), lambda i: (0, 0)),
            pl.BlockSpec((1, _E), lambda i: (0, 0)),
            pl.BlockSpec((_E, _E), lambda i: (0, 0)),
            pl.BlockSpec((1, _E), lambda i: (0, 0)),
        ],
        out_specs=pl.BlockSpec((BB, _E), lambda i: (i, 0)),
        out_shape=jax.ShapeDtypeStruct((_B, _E), jnp.float32),
    )(g, X_num, W_num, b_num2, WnT, b_final2)


def kernel(X_cat, X_num, tables, W_num, b_num, W_final, b_final):
    xc = X_cat.astype(jnp.int32)
    fcol = jnp.arange(_F, dtype=jnp.int32) // 2
    pair_idx = (xc + (fcol * _VP)[None, :]).reshape(_NW, _J, 128)
    pair_idx = jnp.pad(pair_idx, ((0, 0), (0, _JP - _J), (0, 0)))

    tabT = jnp.transpose(tables, (0, 2, 1))
    Wc3 = jnp.transpose(W_final[:, : _F * _E].reshape(_E, _F, _E), (1, 2, 0))
    Wc4 = Wc3.reshape(13, 2, _E, _E)

    tab2 = _tc_transform(tabT, Wc4)
    gathered = _sc_gather_fn()(pair_idx, tab2)
    g = gathered.reshape(_B, _F, 128)

    WnT = W_final[:, _F * _E:].T
    return _tc_combine(g, X_num, W_num, b_num.reshape(1, _E),
                       WnT, b_final.reshape(1, _E))


# speedup vs baseline: 2.3479x; 1.2171x over previous
import functools

import jax
import jax.numpy as jnp
from jax import lax
from jax.experimental import pallas as pl
from jax.experimental.pallas import tpu as pltpu
from jax.experimental.pallas import tpu_sc as plsc

_B, _F, _V, _E, _NUM = 4096, 26, 100000, 64, 13
_D = _F * _E + _E
_NC, _NS = 2, 16
_NW = _NC * _NS
_RPW = _B * _F // _NW
_J = _RPW // 128
_JP = 32
_GV = 4096
_NV = -(-_V // _GV)
_VP = _NV * _GV


def _transform_body(t_ref, wp_ref, out_ref):
    lhs = t_ref[...].reshape(2 * _E, _GV)
    out_ref[...] = lax.dot_general(
        lhs, wp_ref[0], (((0,), (0,)), ((), ())),
        preferred_element_type=jnp.float32)


def _tc_transform(tabT, Wp):
    return pl.pallas_call(
        _transform_body,
        grid=(13, _NV),
        in_specs=[
            pl.BlockSpec((2, _E, _GV), lambda k, j: (k, 0, j)),
            pl.BlockSpec((1, 2 * _E, 2 * _E), lambda k, j: (k, 0, 0)),
        ],
        out_specs=pl.BlockSpec((_GV, 2 * _E), lambda k, j: (k * _NV + j, 0)),
        out_shape=jax.ShapeDtypeStruct((13 * _VP, 2 * _E), jnp.float32),
    )(tabT, Wp)


def _make_sc_gather():
    mesh = plsc.VectorSubcoreMesh(
        core_axis_name="c", subcore_axis_name="s",
        num_cores=_NC, num_subcores=_NS)

    @functools.partial(
        pl.kernel,
        out_type=jax.ShapeDtypeStruct((_NW, _J, 128, 128), jnp.float32),
        mesh=mesh,
        scratch_types=[
            pltpu.VMEM((_JP, 128), jnp.int32),
            pltpu.VMEM((128, 128), jnp.float32),
            pltpu.SemaphoreType.DMA,
        ],
    )
    def sc_gather(idx_hbm, tab_hbm, out_hbm, idx_v, rows_v, sem):
        wid = lax.axis_index("s") * _NC + lax.axis_index("c")
        pltpu.sync_copy(idx_hbm.at[wid], idx_v)

        def body(j, carry):
            pltpu.async_copy(tab_hbm.at[idx_v.at[j]], rows_v, sem).wait()
            pltpu.sync_copy(rows_v, out_hbm.at[wid, j])
            return carry

        lax.fori_loop(0, _J, body, 0, unroll=False)

    return sc_gather


_SC_GATHER_CACHE = []


def _sc_gather_fn():
    if not _SC_GATHER_CACHE:
        _SC_GATHER_CACHE.append(_make_sc_gather())
    return _SC_GATHER_CACHE[0]


def _tc_combine_body(g_ref, xn_ref, wn_ref, bn_ref, wnt_ref, bf_ref, out_ref):
    num_emb = lax.dot_general(
        xn_ref[...], wn_ref[...], (((1,), (1,)), ((), ())),
        preferred_element_type=jnp.float32) + bn_ref[...]
    o = bf_ref[...] + lax.dot_general(
        num_emb, wnt_ref[...], (((1,), (0,)), ((), ())),
        preferred_element_type=jnp.float32)
    g = g_ref[...]
    for f in range(_F):
        o = o + (g[:, f, :_E] if f % 2 == 0 else g[:, f, _E:])
    out_ref[...] = o


def _tc_combine(g, X_num, W_num, b_num2, WnT, b_final2):
    BB = 512
    return pl.pallas_call(
        _tc_combine_body,
        grid=(_B // BB,),
        in_specs=[
            pl.BlockSpec((BB, _F, 128), lambda i: (i, 0, 0)),
            pl.BlockSpec((BB, _NUM), lambda i: (i, 0)),
            pl.BlockSpec((_E, _NUM), lambda i: (0, 0)),
            pl.BlockSpec((1, _E), lambda i: (0, 0)),
            pl.BlockSpec((_E, _E), lambda i: (0, 0)),
            pl.BlockSpec((1, _E), lambda i: (0, 0)),
        ],
        out_specs=pl.BlockSpec((BB, _E), lambda i: (i, 0)),
        out_shape=jax.ShapeDtypeStruct((_B, _E), jnp.float32),
    )(g, X_num, W_num, b_num2, WnT, b_final2)


def kernel(X_cat, X_num, tables, W_num, b_num, W_final, b_final):
    xc = X_cat.astype(jnp.int32)
    fcol = jnp.arange(_F, dtype=jnp.int32) // 2
    pair_idx = (xc + (fcol * _VP)[None, :]).reshape(_NW, _J, 128)
    pair_idx = jnp.pad(pair_idx, ((0, 0), (0, _JP - _J), (0, 0)))

    tabT = jnp.transpose(tables, (0, 2, 1))
    Wc3 = jnp.transpose(W_final[:, : _F * _E].reshape(_E, _F, _E), (1, 2, 0))
    Wp = jnp.zeros((13, 2 * _E, 2 * _E), jnp.float32)
    Wp = Wp.at[:, :_E, :_E].set(Wc3[0::2]).at[:, _E:, _E:].set(Wc3[1::2])

    tab2 = _tc_transform(tabT, Wp)
    gathered = _sc_gather_fn()(pair_idx, tab2)
    g = gathered.reshape(_B, _F, 128)

    WnT = W_final[:, _F * _E:].T
    return _tc_combine(g, X_num, W_num, b_num.reshape(1, _E),
                       WnT, b_final.reshape(1, _E))


# speedup vs baseline: 2.8988x; 1.2346x over previous
import functools

import jax
import jax.numpy as jnp
from jax import lax
from jax.experimental import pallas as pl
from jax.experimental.pallas import tpu as pltpu
from jax.experimental.pallas import tpu_sc as plsc

_B, _F, _V, _E, _NUM = 4096, 26, 100000, 64, 13
_D = _F * _E + _E
_NC, _NS = 2, 16
_NW = _NC * _NS
_RPW = _B * _F // _NW
_J = _RPW // 128
_JP = 32
_GV = 8192
_NV = -(-_V // _GV)
_VP = _NV * _GV


def _transform_body(t_ref, wp_ref, out_ref):
    lhs = t_ref[...].reshape(2 * _E, _GV)
    out_ref[...] = lax.dot_general(
        lhs, wp_ref[0], (((0,), (0,)), ((), ())),
        preferred_element_type=jnp.float32)


def _tc_transform(tabT, Wp):
    return pl.pallas_call(
        _transform_body,
        grid=(13, _NV),
        in_specs=[
            pl.BlockSpec((2, _E, _GV), lambda k, j: (k, 0, j)),
            pl.BlockSpec((1, 2 * _E, 2 * _E), lambda k, j: (k, 0, 0)),
        ],
        out_specs=pl.BlockSpec((_GV, 2 * _E), lambda k, j: (k * _NV + j, 0)),
        out_shape=jax.ShapeDtypeStruct((13 * _VP, 2 * _E), jnp.float32),
    )(tabT, Wp)


def _make_sc_gather():
    mesh = plsc.VectorSubcoreMesh(
        core_axis_name="c", subcore_axis_name="s",
        num_cores=_NC, num_subcores=_NS)

    @functools.partial(
        pl.kernel,
        out_type=jax.ShapeDtypeStruct((_NW, _J, 128, 128), jnp.float32),
        mesh=mesh,
        scratch_types=[
            pltpu.VMEM((_JP, 128), jnp.int32),
            pltpu.VMEM((128, 128), jnp.float32),
            pltpu.SemaphoreType.DMA,
        ],
    )
    def sc_gather(idx_hbm, tab_hbm, out_hbm, idx_v, rows_v, sem):
        wid = lax.axis_index("s") * _NC + lax.axis_index("c")
        pltpu.sync_copy(idx_hbm.at[wid], idx_v)

        def body(j, carry):
            pltpu.async_copy(tab_hbm.at[idx_v.at[j]], rows_v, sem).wait()
            pltpu.sync_copy(rows_v, out_hbm.at[wid, j])
            return carry

        lax.fori_loop(0, _J, body, 0, unroll=False)

    return sc_gather


_SC_GATHER_CACHE = []


def _sc_gather_fn():
    if not _SC_GATHER_CACHE:
        _SC_GATHER_CACHE.append(_make_sc_gather())
    return _SC_GATHER_CACHE[0]


def _tc_combine_body(g_ref, xn_ref, wn_ref, bn_ref, wnt_ref, bf_ref, out_ref):
    num_emb = lax.dot_general(
        xn_ref[...], wn_ref[...], (((1,), (1,)), ((), ())),
        preferred_element_type=jnp.float32) + bn_ref[...]
    o = bf_ref[...] + lax.dot_general(
        num_emb, wnt_ref[...], (((1,), (0,)), ((), ())),
        preferred_element_type=jnp.float32)
    g = g_ref[...]
    for f in range(_F):
        o = o + (g[0, f, :, :_E] if f % 2 == 0 else g[0, f, :, _E:])
    out_ref[...] = o


def _tc_combine(g, X_num, W_num, b_num2, WnT, b_final2):
    BB = _B // _NW
    return pl.pallas_call(
        _tc_combine_body,
        grid=(_B // BB,),
        in_specs=[
            pl.BlockSpec((1, _J, 128, 128), lambda i: (i, 0, 0, 0)),
            pl.BlockSpec((BB, _NUM), lambda i: (i, 0)),
            pl.BlockSpec((_E, _NUM), lambda i: (0, 0)),
            pl.BlockSpec((1, _E), lambda i: (0, 0)),
            pl.BlockSpec((_E, _E), lambda i: (0, 0)),
            pl.BlockSpec((1, _E), lambda i: (0, 0)),
        ],
        out_specs=pl.BlockSpec((BB, _E), lambda i: (i, 0)),
        out_shape=jax.ShapeDtypeStruct((_B, _E), jnp.float32),
    )(g, X_num, W_num, b_num2, WnT, b_final2)


def kernel(X_cat, X_num, tables, W_num, b_num, W_final, b_final):
    xc = X_cat.astype(jnp.int32)
    fcol = jnp.arange(_F, dtype=jnp.int32) // 2
    pair_idx = (xc + (fcol * _VP)[None, :])
    pair_idx = jnp.transpose(pair_idx.reshape(_NW, _B // _NW, _F), (0, 2, 1))
    pair_idx = jnp.pad(pair_idx, ((0, 0), (0, _JP - _J), (0, 0)))

    tabT = jnp.transpose(tables, (0, 2, 1))
    Wc3 = jnp.transpose(W_final[:, : _F * _E].reshape(_E, _F, _E), (1, 2, 0))
    Wp = jnp.zeros((13, 2 * _E, 2 * _E), jnp.float32)
    Wp = Wp.at[:, :_E, :_E].set(Wc3[0::2]).at[:, _E:, _E:].set(Wc3[1::2])

    tab2 = _tc_transform(tabT, Wp)
    g = _sc_gather_fn()(pair_idx, tab2)

    WnT = W_final[:, _F * _E:].T
    return _tc_combine(g, X_num, W_num, b_num.reshape(1, _E),
                       WnT, b_final.reshape(1, _E))


# speedup vs baseline: 3.0163x; 1.0405x over previous
import functools

import jax
import jax.numpy as jnp
from jax import lax
from jax.experimental import pallas as pl
from jax.experimental.pallas import tpu as pltpu
from jax.experimental.pallas import tpu_sc as plsc

_B, _F, _V, _E, _NUM = 4096, 26, 100000, 64, 13
_D = _F * _E + _E
_NC, _NS = 2, 16
_NW = _NC * _NS
_RPW = _B * _F // _NW
_J = _RPW // 128
_JP = 32
_GV = 12800
_NV = -(-_V // _GV)
_VP = _NV * _GV


def _transform_body(t_ref, wp_ref, out_ref):
    lhs = t_ref[...].reshape(2 * _E, _GV)
    out_ref[...] = lax.dot_general(
        lhs, wp_ref[0], (((0,), (0,)), ((), ())),
        preferred_element_type=jnp.float32)


def _tc_transform(tabT, Wp):
    return pl.pallas_call(
        _transform_body,
        grid=(13, _NV),
        in_specs=[
            pl.BlockSpec((2, _E, _GV), lambda k, j: (k, 0, j)),
            pl.BlockSpec((1, 2 * _E, 2 * _E), lambda k, j: (k, 0, 0)),
        ],
        out_specs=pl.BlockSpec((_GV, 2 * _E), lambda k, j: (k * _NV + j, 0)),
        out_shape=jax.ShapeDtypeStruct((13 * _VP, 2 * _E), jnp.float32),
    )(tabT, Wp)


def _make_sc_gather():
    mesh = plsc.VectorSubcoreMesh(
        core_axis_name="c", subcore_axis_name="s",
        num_cores=_NC, num_subcores=_NS)

    @functools.partial(
        pl.kernel,
        out_type=jax.ShapeDtypeStruct((_NW, _J, 128, 128), jnp.float32),
        mesh=mesh,
        scratch_types=[
            pltpu.VMEM((_JP, 128), jnp.int32),
            pltpu.VMEM((128, 128), jnp.float32),
            pltpu.SemaphoreType.DMA,
        ],
    )
    def sc_gather(idx_hbm, tab_hbm, out_hbm, idx_v, rows_v, sem):
        wid = lax.axis_index("s") * _NC + lax.axis_index("c")
        pltpu.sync_copy(idx_hbm.at[wid], idx_v)

        def body(j, carry):
            pltpu.async_copy(tab_hbm.at[idx_v.at[j]], rows_v, sem).wait()
            pltpu.sync_copy(rows_v, out_hbm.at[wid, j])
            return carry

        lax.fori_loop(0, _J, body, 0, unroll=False)

    return sc_gather


_SC_GATHER_CACHE = []


def _sc_gather_fn():
    if not _SC_GATHER_CACHE:
        _SC_GATHER_CACHE.append(_make_sc_gather())
    return _SC_GATHER_CACHE[0]


def _tc_combine_body(g_ref, xn_ref, wn_ref, bn_ref, wnt_ref, bf_ref, out_ref):
    num_emb = lax.dot_general(
        xn_ref[...], wn_ref[...], (((1,), (1,)), ((), ())),
        preferred_element_type=jnp.float32) + bn_ref[...]
    o = bf_ref[...] + lax.dot_general(
        num_emb, wnt_ref[...], (((1,), (0,)), ((), ())),
        preferred_element_type=jnp.float32)
    g = g_ref[...]
    for f in range(_F):
        o = o + (g[0, f, :, :_E] if f % 2 == 0 else g[0, f, :, _E:])
    out_ref[...] = o


def _tc_combine(g, X_num, W_num, b_num2, WnT, b_final2):
    BB = _B // _NW
    return pl.pallas_call(
        _tc_combine_body,
        grid=(_B // BB,),
        in_specs=[
            pl.BlockSpec((1, _J, 128, 128), lambda i: (i, 0, 0, 0)),
            pl.BlockSpec((BB, _NUM), lambda i: (i, 0)),
            pl.BlockSpec((_E, _NUM), lambda i: (0, 0)),
            pl.BlockSpec((1, _E), lambda i: (0, 0)),
            pl.BlockSpec((_E, _E), lambda i: (0, 0)),
            pl.BlockSpec((1, _E), lambda i: (0, 0)),
        ],
        out_specs=pl.BlockSpec((BB, _E), lambda i: (i, 0)),
        out_shape=jax.ShapeDtypeStruct((_B, _E), jnp.float32),
    )(g, X_num, W_num, b_num2, WnT, b_final2)


def kernel(X_cat, X_num, tables, W_num, b_num, W_final, b_final):
    xc = X_cat.astype(jnp.int32)
    fcol = jnp.arange(_F, dtype=jnp.int32) // 2
    pair_idx = (xc + (fcol * _VP)[None, :])
    pair_idx = jnp.transpose(pair_idx.reshape(_NW, _B // _NW, _F), (0, 2, 1))
    pair_idx = jnp.pad(pair_idx, ((0, 0), (0, _JP - _J), (0, 0)))

    tabT = jnp.transpose(tables, (0, 2, 1))
    Wc3 = jnp.transpose(W_final[:, : _F * _E].reshape(_E, _F, _E), (1, 2, 0))
    Wp = jnp.zeros((13, 2 * _E, 2 * _E), jnp.float32)
    Wp = Wp.at[:, :_E, :_E].set(Wc3[0::2]).at[:, _E:, _E:].set(Wc3[1::2])

    tab2 = _tc_transform(tabT, Wp)
    g = _sc_gather_fn()(pair_idx, tab2)

    WnT = W_final[:, _F * _E:].T
    return _tc_combine(g, X_num, W_num, b_num.reshape(1, _E),
                       WnT, b_final.reshape(1, _E))


# speedup vs baseline: 3.5016x; 1.1609x over previous
import functools

import jax
import jax.numpy as jnp
from jax import lax
from jax.experimental import pallas as pl
from jax.experimental.pallas import tpu as pltpu
from jax.experimental.pallas import tpu_sc as plsc

_B, _F, _V, _E, _NUM = 4096, 26, 100000, 64, 13
_D = _F * _E + _E
_NC, _NS = 2, 16
_NW = _NC * _NS
_RPW = _B * _F // _NW
_J = _RPW // 128
_JP = 32
_GV = 12800
_NV = -(-_V // _GV)
_VP = _NV * _GV


def _transform_body(t_ref, wp_ref, out_ref):
    lhs = t_ref[...].reshape(2 * _E, _GV)
    r = lax.dot_general(
        lhs, wp_ref[0], (((0,), (0,)), ((), ())),
        preferred_element_type=jnp.float32)
    au = lax.bitcast_convert_type(
        r[: _GV // 2].astype(jnp.bfloat16), jnp.uint16).astype(jnp.uint32)
    bu = lax.bitcast_convert_type(
        r[_GV // 2:].astype(jnp.bfloat16), jnp.uint16).astype(jnp.uint32)
    out_ref[...] = lax.bitcast_convert_type((au << 16) | bu, jnp.float32)


def _tc_transform(tabT, Wp):
    return pl.pallas_call(
        _transform_body,
        grid=(13, _NV),
        in_specs=[
            pl.BlockSpec((2, _E, _GV), lambda k, j: (k, 0, j)),
            pl.BlockSpec((1, 2 * _E, 2 * _E), lambda k, j: (k, 0, 0)),
        ],
        out_specs=pl.BlockSpec((_GV // 2, 2 * _E),
                               lambda k, j: (k * _NV + j, 0)),
        out_shape=jax.ShapeDtypeStruct((13 * _VP // 2, 2 * _E), jnp.float32),
    )(tabT, Wp)


def _make_sc_gather():
    mesh = plsc.VectorSubcoreMesh(
        core_axis_name="c", subcore_axis_name="s",
        num_cores=_NC, num_subcores=_NS)

    @functools.partial(
        pl.kernel,
        out_type=jax.ShapeDtypeStruct((_NW, _J, 128, 128), jnp.float32),
        mesh=mesh,
        scratch_types=[
            pltpu.VMEM((_JP, 128), jnp.int32),
            pltpu.VMEM((128, 128), jnp.float32),
            pltpu.SemaphoreType.DMA,
        ],
    )
    def sc_gather(idx_hbm, tab_hbm, out_hbm, idx_v, rows_v, sem):
        wid = lax.axis_index("s") * _NC + lax.axis_index("c")
        pltpu.sync_copy(idx_hbm.at[wid], idx_v)

        def body(j, carry):
            pltpu.async_copy(tab_hbm.at[idx_v.at[j]], rows_v, sem).wait()
            pltpu.sync_copy(rows_v, out_hbm.at[wid, j])
            return carry

        lax.fori_loop(0, _J, body, 0, unroll=False)

    return sc_gather


_SC_GATHER_CACHE = []


def _sc_gather_fn():
    if not _SC_GATHER_CACHE:
        _SC_GATHER_CACHE.append(_make_sc_gather())
    return _SC_GATHER_CACHE[0]


def _tc_combine_body(g_ref, sel_ref, xn_ref, wn_ref, bn_ref, wnt_ref,
                     bf_ref, out_ref):
    num_emb = lax.dot_general(
        xn_ref[...], wn_ref[...], (((1,), (1,)), ((), ())),
        preferred_element_type=jnp.float32) + bn_ref[...]
    o = bf_ref[...] + lax.dot_general(
        num_emb, wnt_ref[...], (((1,), (0,)), ((), ())),
        preferred_element_type=jnp.float32)
    g = lax.bitcast_convert_type(g_ref[...], jnp.uint32)
    sel = sel_ref[...]
    for f in range(_F):
        u = g[0, f]
        even = lax.bitcast_convert_type(u & jnp.uint32(0xFFFF0000),
                                        jnp.float32)
        odd = lax.bitcast_convert_type(u << 16, jnp.float32)
        m = sel[0, f][:, None] > 0.5
        row = jnp.where(m, odd, even)
        o = o + (row[:, :_E] if f % 2 == 0 else row[:, _E:])
    out_ref[...] = o


def _tc_combine(g, sel, X_num, W_num, b_num2, WnT, b_final2):
    BB = _B // _NW
    return pl.pallas_call(
        _tc_combine_body,
        grid=(_B // BB,),
        in_specs=[
            pl.BlockSpec((1, _J, 128, 128), lambda i: (i, 0, 0, 0)),
            pl.BlockSpec((1, _J, 128), lambda i: (i, 0, 0)),
            pl.BlockSpec((BB, _NUM), lambda i: (i, 0)),
            pl.BlockSpec((_E, _NUM), lambda i: (0, 0)),
            pl.BlockSpec((1, _E), lambda i: (0, 0)),
            pl.BlockSpec((_E, _E), lambda i: (0, 0)),
            pl.BlockSpec((1, _E), lambda i: (0, 0)),
        ],
        out_specs=pl.BlockSpec((BB, _E), lambda i: (i, 0)),
        out_shape=jax.ShapeDtypeStruct((_B, _E), jnp.float32),
    )(g, sel, X_num, W_num, b_num2, WnT, b_final2)


def kernel(X_cat, X_num, tables, W_num, b_num, W_final, b_final):
    xc = X_cat.astype(jnp.int32)
    fcol = jnp.arange(_F, dtype=jnp.int32) // 2
    h = _GV // 2
    pair_idx = ((fcol * _NV)[None, :] + xc // _GV) * h + (xc % h)
    pair_idx = jnp.transpose(pair_idx.reshape(_NW, _B // _NW, _F), (0, 2, 1))
    pair_idx = jnp.pad(pair_idx, ((0, 0), (0, _JP - _J), (0, 0)))
    sel = jnp.transpose(((xc // h) & 1).astype(jnp.float32).reshape(
        _NW, _B // _NW, _F), (0, 2, 1))

    tabT = jnp.transpose(tables, (0, 2, 1))
    Wc3 = jnp.transpose(W_final[:, : _F * _E].reshape(_E, _F, _E), (1, 2, 0))
    Wp = jnp.zeros((13, 2 * _E, 2 * _E), jnp.float32)
    Wp = Wp.at[:, :_E, :_E].set(Wc3[0::2]).at[:, _E:, _E:].set(Wc3[1::2])

    tab2 = _tc_transform(tabT, Wp)
    g = _sc_gather_fn()(pair_idx, tab2)

    WnT = W_final[:, _F * _E:].T
    return _tc_combine(g, sel, X_num, W_num, b_num.reshape(1, _E),
                       WnT, b_final.reshape(1, _E))
